# baseline (speedup 1.0000x reference)
import functools

import jax
import jax.numpy as jnp
from jax import lax
from jax.experimental import pallas as pl
from jax.experimental.pallas import tpu as pltpu
from jax.experimental.pallas import tpu_sc as plsc

N = 10000
NP = 10240
E = 160000
NC = 2
NS = 16
ROWS_PER_TILE = NP // NS
CHUNK = 128

EDGES_PER_TILE_AGG = E // NS
AGG_FULL_CHUNKS = EDGES_PER_TILE_AGG // CHUNK
AGG_TAIL = EDGES_PER_TILE_AGG - AGG_FULL_CHUNKS * CHUNK

EDGES_PER_TILE_DEG = E // (NC * NS)
DEG_FULL_CHUNKS = EDGES_PER_TILE_DEG // CHUNK
DEG_TAIL = EDGES_PER_TILE_DEG - DEG_FULL_CHUNKS * CHUNK


def _mesh():
    return plsc.VectorSubcoreMesh(core_axis_name="c", subcore_axis_name="s")


@functools.partial(
    pl.kernel,
    out_type=jax.ShapeDtypeStruct((NC, NP, 16), jnp.float32),
    mesh=_mesh(),
    scratch_types=[
        pltpu.VMEM((2, CHUNK), jnp.int32),
        pltpu.VMEM((DEG_TAIL,), jnp.int32),
        pltpu.VMEM((CHUNK, 16), jnp.float32),
        pltpu.VMEM((ROWS_PER_TILE, 16), jnp.float32),
        pltpu.VMEM_SHARED((NP, 16), jnp.float32),
        pltpu.SemaphoreType.DMA,
        pltpu.SemaphoreType.DMA,
    ],
    compiler_params=pltpu.CompilerParams(use_tc_tiling_on_sc=False),
)
def _sc_degree(dst_hbm, ones_hbm, zeros_hbm, out_hbm, dstv, dstv_t, onesv,
               zerov, acc, sd0, sd1):
    c = lax.axis_index("c")
    s = lax.axis_index("s")
    sems = (sd0, sd1)

    pltpu.sync_copy(ones_hbm, onesv)
    pltpu.sync_copy(zeros_hbm, zerov)

    r0 = s * ROWS_PER_TILE
    pltpu.sync_copy(zerov, acc.at[pl.ds(r0, ROWS_PER_TILE)])
    plsc.subcore_barrier()

    base = c * (E // NC) + s * EDGES_PER_TILE_DEG

    def fetch_d(buf, j):
        b = pl.multiple_of(base + j * CHUNK, 8)
        pltpu.async_copy(dst_hbm.at[pl.ds(b, CHUNK)], dstv.at[buf], sems[buf])

    def drain_d(buf):
        pltpu.make_async_copy(dst_hbm.at[pl.ds(0, CHUNK)], dstv.at[buf],
                              sems[buf]).wait()
        pltpu.sync_copy(onesv, acc.at[dstv.at[buf]], add=True)

    fetch_d(0, 0)

    @pl.loop(0, DEG_FULL_CHUNKS - 1, step=2)
    def _(j):
        fetch_d(1, j + 1)
        drain_d(0)
        fetch_d(0, j + 2)
        drain_d(1)

    drain_d(0)

    bt = pl.multiple_of(base + DEG_FULL_CHUNKS * CHUNK, 8)
    pltpu.sync_copy(dst_hbm.at[pl.ds(bt, DEG_TAIL)], dstv_t)
    pltpu.sync_copy(onesv.at[pl.ds(0, DEG_TAIL)], acc.at[dstv_t], add=True)

    plsc.subcore_barrier()
    pltpu.sync_copy(acc.at[pl.ds(r0, ROWS_PER_TILE)],
                    out_hbm.at[c, pl.ds(r0, ROWS_PER_TILE)])


def _make_sc_agg(HD):
    @functools.partial(
        pl.kernel,
        out_type=jax.ShapeDtypeStruct((2 * NP, HD), jnp.float32),
        mesh=_mesh(),
        scratch_types=[
            pltpu.VMEM((3, 2, CHUNK), jnp.int32),
            pltpu.VMEM((2, CHUNK), jnp.int32),
            pltpu.VMEM((2, CHUNK), jnp.int32),
            pltpu.VMEM((2, CHUNK, HD), jnp.float32),
            pltpu.VMEM((AGG_TAIL,), jnp.int32),
            pltpu.VMEM((AGG_TAIL,), jnp.int32),
            pltpu.VMEM((AGG_TAIL,), jnp.int32),
            pltpu.VMEM((AGG_TAIL, HD), jnp.float32),
            pltpu.VMEM_SHARED((NP, HD), jnp.float32),
            pltpu.SemaphoreType.DMA,
            pltpu.SemaphoreType.DMA,
            pltpu.SemaphoreType.DMA,
            pltpu.SemaphoreType.DMA,
            pltpu.SemaphoreType.DMA,
            pltpu.SemaphoreType.DMA,
            pltpu.SemaphoreType.DMA,
        ],
        compiler_params=pltpu.CompilerParams(use_tc_tiling_on_sc=False),
    )
    def agg(h_hbm, ei_hbm, out_hbm,
            eiv, idxv, dstv, rowsv, srcv_t, dstv_t, idxv_t, rowsv_t,
            acc, se0, se1, se2, sg0, sg1, ss0, ss1):
        c = lax.axis_index("c")
        s = lax.axis_index("s")
        r0 = s * ROWS_PER_TILE
        row_off = c * NP
        sems_e = (se0, se1, se2)
        sems_g = (sg0, sg1)
        sems_s = (ss0, ss1)

        pltpu.sync_copy(h_hbm.at[pl.ds(row_off + r0, ROWS_PER_TILE)],
                        acc.at[pl.ds(r0, ROWS_PER_TILE)])
        plsc.subcore_barrier()

        ebase = s * EDGES_PER_TILE_AGG

        def fetch_ei(k, e):
            b = pl.multiple_of(ebase + k * CHUNK, 8)
            pltpu.async_copy(ei_hbm.at[:, pl.ds(b, CHUNK)], eiv.at[e],
                             sems_e[e])

        def wait_scatter(g):
            pltpu.make_async_copy(rowsv.at[g], acc.at[dstv.at[g]],
                                  sems_s[g]).wait()

        def start_gather(e, g, prior_scatter=None):
            pltpu.make_async_copy(ei_hbm.at[:, pl.ds(0, CHUNK)],
                                  eiv.at[e], sems_e[e]).wait()
            if prior_scatter is not None:
                @pl.when(prior_scatter)
                def _():
                    wait_scatter(g)
            for i in range(CHUNK // 16):
                sl = pl.ds(i * 16, 16)
                idxv[g, sl] = eiv[e, 0, sl] + row_off
                dstv[g, sl] = eiv[e, 1, sl]
            pltpu.async_copy(h_hbm.at[idxv.at[g]], rowsv.at[g], sems_g[g])

        def scatter(g):
            pltpu.make_async_copy(h_hbm.at[idxv.at[g]], rowsv.at[g],
                                  sems_g[g]).wait()
            pltpu.async_copy(rowsv.at[g], acc.at[dstv.at[g]], sems_s[g],
                             add=True)

        fetch_ei(0, 0)
        fetch_ei(1, 1)
        start_gather(0, 0)

        @pl.loop(0, AGG_FULL_CHUNKS, step=6)
        def _(j):
            for b in range(6):
                k = j + b

                @pl.when(k + 2 < AGG_FULL_CHUNKS)
                def _():
                    fetch_ei(k + 2, (b + 2) % 3)

                @pl.when(k + 1 < AGG_FULL_CHUNKS)
                def _():
                    start_gather((b + 1) % 3, (b + 1) % 2,
                                 prior_scatter=(k + 1 >= 2))

                scatter(b % 2)

        wait_scatter(0)
        wait_scatter(1)

        bt = pl.multiple_of(ebase + AGG_FULL_CHUNKS * CHUNK, 8)
        pltpu.sync_copy(ei_hbm.at[0, pl.ds(bt, AGG_TAIL)], srcv_t)
        pltpu.sync_copy(ei_hbm.at[1, pl.ds(bt, AGG_TAIL)], dstv_t)
        idxv_t[...] = srcv_t[...] + row_off
        pltpu.async_copy(h_hbm.at[idxv_t], rowsv_t, se0).wait()
        pltpu.sync_copy(rowsv_t, acc.at[dstv_t], add=True)

        plsc.subcore_barrier()
        pltpu.sync_copy(acc.at[pl.ds(r0, ROWS_PER_TILE)],
                        out_hbm.at[pl.ds(row_off + r0, ROWS_PER_TILE)])

    return agg


_sc_agg128 = _make_sc_agg(128)
_sc_agg32 = _make_sc_agg(32)


def _split_store(out_ref, h):
    hw = h.shape[1] // 2
    out_ref[0:N, :] = h[:, 0:hw]
    out_ref[NP:NP + N, :] = h[:, hw:]
    pad = jnp.zeros((NP - N, hw), jnp.float32)
    out_ref[N:NP, :] = pad
    out_ref[NP + N:2 * NP, :] = pad


def _dinv(degp_ref):
    deg = degp_ref[0, 0:N, 0:1] + degp_ref[1, 0:N, 0:1] + 1.0
    return lax.rsqrt(deg)


def _tc1_body(degp_ref, x_ref, w_ref, out_ref):
    dinv = _dinv(degp_ref)
    h = jnp.dot(x_ref[...], w_ref[...], preferred_element_type=jnp.float32)
    h = h * dinv
    _split_store(out_ref, h)


def _tc_mid_body(degp_ref, agg_ref, b_ref, g_ref, be_ref, w_ref, out_ref):
    dinv = _dinv(degp_ref)
    z = jnp.concatenate([agg_ref[0:N, :], agg_ref[NP:NP + N, :]], axis=1)
    z = z * dinv + b_ref[...]
    z = jnp.maximum(z, 0.0)
    mean = jnp.mean(z, axis=0, keepdims=True)
    var = jnp.mean((z - mean) * (z - mean), axis=0, keepdims=True)
    z = g_ref[...] * (z - mean) * lax.rsqrt(var + 1e-5) + be_ref[...]
    h = jnp.dot(z, w_ref[...], preferred_element_type=jnp.float32)
    h = h * dinv
    _split_store(out_ref, h)


def _tc_final_body(degp_ref, agg_ref, b_ref, out_ref):
    dinv = _dinv(degp_ref)
    z = jnp.concatenate([agg_ref[0:N, :], agg_ref[NP:NP + N, :]], axis=1)
    z = z * dinv + b_ref[...]
    m = jnp.max(z, axis=1, keepdims=True)
    zm = z - m
    lse = jnp.log(jnp.sum(jnp.exp(zm), axis=1, keepdims=True))
    out_ref[...] = zm - lse


def _tc_call(body, out_shape, *args):
    return pl.pallas_call(
        body, out_shape=jax.ShapeDtypeStruct(out_shape, jnp.float32))(*args)


def kernel(x, edge_index, W1, b1, W2, b2, W3, b3, gamma1, beta1, gamma2, beta2):
    src = edge_index[0]
    dst = edge_index[1]

    ones16 = jnp.ones((CHUNK, 16), jnp.float32)
    zeros16 = jnp.zeros((ROWS_PER_TILE, 16), jnp.float32)
    degp = _sc_degree(dst, ones16, zeros16)
    h1 = _tc_call(_tc1_body, (2 * NP, 128), degp, x, W1)
    a1 = _sc_agg128(h1, edge_index)
    h2 = _tc_call(_tc_mid_body, (2 * NP, 128), degp, a1,
                  b1.reshape(1, -1), gamma1.reshape(1, -1),
                  beta1.reshape(1, -1), W2)
    a2 = _sc_agg128(h2, edge_index)
    h3 = _tc_call(_tc_mid_body, (2 * NP, 32), degp, a2,
                  b2.reshape(1, -1), gamma2.reshape(1, -1),
                  beta2.reshape(1, -1), W3)
    a3 = _sc_agg32(h3, edge_index)
    out = _tc_call(_tc_final_body, (N, 64), degp, a3, b3.reshape(1, -1))
    return out


# speedup vs baseline: 1.0394x; 1.0394x over previous
import functools

import jax
import jax.numpy as jnp
from jax import lax
from jax.experimental import pallas as pl
from jax.experimental.pallas import tpu as pltpu
from jax.experimental.pallas import tpu_sc as plsc

N = 10000
NP = 10240
E = 160000
NC = 2
NS = 16
ROWS_PER_TILE = NP // NS
CHUNK = 128

EDGES_PER_TILE_AGG = E // NS
AGG_FULL_CHUNKS = EDGES_PER_TILE_AGG // CHUNK
AGG_TAIL = EDGES_PER_TILE_AGG - AGG_FULL_CHUNKS * CHUNK

EDGES_PER_TILE_DEG = E // (NC * NS)
DEG_FULL_CHUNKS = EDGES_PER_TILE_DEG // CHUNK
DEG_TAIL = EDGES_PER_TILE_DEG - DEG_FULL_CHUNKS * CHUNK


def _mesh():
    return plsc.VectorSubcoreMesh(core_axis_name="c", subcore_axis_name="s")


@functools.partial(
    pl.kernel,
    out_type=jax.ShapeDtypeStruct((NC, NP, 16), jnp.float32),
    mesh=_mesh(),
    scratch_types=[
        pltpu.VMEM((2, CHUNK), jnp.int32),
        pltpu.VMEM((DEG_TAIL,), jnp.int32),
        pltpu.VMEM((CHUNK, 16), jnp.float32),
        pltpu.VMEM((ROWS_PER_TILE, 16), jnp.float32),
        pltpu.VMEM_SHARED((NP, 16), jnp.float32),
        pltpu.SemaphoreType.DMA,
        pltpu.SemaphoreType.DMA,
    ],
    compiler_params=pltpu.CompilerParams(use_tc_tiling_on_sc=False),
)
def _sc_degree(dst_hbm, ones_hbm, zeros_hbm, out_hbm, dstv, dstv_t, onesv,
               zerov, acc, sd0, sd1):
    c = lax.axis_index("c")
    s = lax.axis_index("s")
    sems = (sd0, sd1)

    pltpu.sync_copy(ones_hbm, onesv)
    pltpu.sync_copy(zeros_hbm, zerov)

    r0 = s * ROWS_PER_TILE
    pltpu.sync_copy(zerov, acc.at[pl.ds(r0, ROWS_PER_TILE)])
    plsc.subcore_barrier()

    base = c * (E // NC) + s * EDGES_PER_TILE_DEG

    def fetch_d(buf, j):
        b = pl.multiple_of(base + j * CHUNK, 8)
        pltpu.async_copy(dst_hbm.at[pl.ds(b, CHUNK)], dstv.at[buf], sems[buf])

    def drain_d(buf):
        pltpu.make_async_copy(dst_hbm.at[pl.ds(0, CHUNK)], dstv.at[buf],
                              sems[buf]).wait()
        pltpu.sync_copy(onesv, acc.at[dstv.at[buf]], add=True)

    fetch_d(0, 0)

    @pl.loop(0, DEG_FULL_CHUNKS - 1, step=2)
    def _(j):
        fetch_d(1, j + 1)
        drain_d(0)
        fetch_d(0, j + 2)
        drain_d(1)

    drain_d(0)

    bt = pl.multiple_of(base + DEG_FULL_CHUNKS * CHUNK, 8)
    pltpu.sync_copy(dst_hbm.at[pl.ds(bt, DEG_TAIL)], dstv_t)
    pltpu.sync_copy(onesv.at[pl.ds(0, DEG_TAIL)], acc.at[dstv_t], add=True)

    plsc.subcore_barrier()
    pltpu.sync_copy(acc.at[pl.ds(r0, ROWS_PER_TILE)],
                    out_hbm.at[c, pl.ds(r0, ROWS_PER_TILE)])


def _make_sc_agg(HD):
    @functools.partial(
        pl.kernel,
        out_type=jax.ShapeDtypeStruct((2 * NP, HD), jnp.float32),
        mesh=_mesh(),
        scratch_types=[
            pltpu.VMEM((3, 2, CHUNK), jnp.int32),
            pltpu.VMEM((2, CHUNK), jnp.int32),
            pltpu.VMEM((2, CHUNK), jnp.int32),
            pltpu.VMEM((2, CHUNK, HD), jnp.float32),
            pltpu.VMEM((AGG_TAIL,), jnp.int32),
            pltpu.VMEM((AGG_TAIL,), jnp.int32),
            pltpu.VMEM((AGG_TAIL,), jnp.int32),
            pltpu.VMEM((AGG_TAIL, HD), jnp.float32),
            pltpu.VMEM_SHARED((NP, HD), jnp.float32),
            pltpu.SemaphoreType.DMA,
            pltpu.SemaphoreType.DMA,
            pltpu.SemaphoreType.DMA,
            pltpu.SemaphoreType.DMA,
            pltpu.SemaphoreType.DMA,
            pltpu.SemaphoreType.DMA,
            pltpu.SemaphoreType.DMA,
        ],
        compiler_params=pltpu.CompilerParams(use_tc_tiling_on_sc=False),
    )
    def agg(h_hbm, ei_hbm, out_hbm,
            eiv, idxv, dstv, rowsv, srcv_t, dstv_t, idxv_t, rowsv_t,
            acc, se0, se1, se2, sg0, sg1, ss0, ss1):
        c = lax.axis_index("c")
        s = lax.axis_index("s")
        r0 = s * ROWS_PER_TILE
        row_off = c * NP
        sems_e = (se0, se1, se2)
        sems_g = (sg0, sg1)
        sems_s = (ss0, ss1)

        pltpu.sync_copy(h_hbm.at[pl.ds(row_off + r0, ROWS_PER_TILE)],
                        acc.at[pl.ds(r0, ROWS_PER_TILE)])
        plsc.subcore_barrier()

        ebase = s * EDGES_PER_TILE_AGG

        def fetch_ei(k, e):
            b = pl.multiple_of(ebase + k * CHUNK, 8)
            pltpu.async_copy(ei_hbm.at[:, pl.ds(b, CHUNK)], eiv.at[e],
                             sems_e[e])

        def wait_scatter(g):
            pltpu.make_async_copy(rowsv.at[g], acc.at[dstv.at[g]],
                                  sems_s[g]).wait()

        def start_gather(e, g, prior_scatter=None):
            pltpu.make_async_copy(ei_hbm.at[:, pl.ds(0, CHUNK)],
                                  eiv.at[e], sems_e[e]).wait()
            if prior_scatter is not None:
                @pl.when(prior_scatter)
                def _():
                    wait_scatter(g)
            for i in range(CHUNK // 16):
                sl = pl.ds(i * 16, 16)
                idxv[g, sl] = eiv[e, 0, sl] + row_off
                dstv[g, sl] = eiv[e, 1, sl]
            pltpu.async_copy(h_hbm.at[idxv.at[g]], rowsv.at[g], sems_g[g])

        def scatter(g):
            pltpu.make_async_copy(h_hbm.at[idxv.at[g]], rowsv.at[g],
                                  sems_g[g]).wait()
            pltpu.async_copy(rowsv.at[g], acc.at[dstv.at[g]], sems_s[g],
                             add=True)

        fetch_ei(0, 0)
        fetch_ei(1, 1)
        start_gather(0, 0)

        @pl.loop(0, AGG_FULL_CHUNKS, step=6)
        def _(j):
            for b in range(6):
                k = j + b

                @pl.when(k + 2 < AGG_FULL_CHUNKS)
                def _():
                    fetch_ei(k + 2, (b + 2) % 3)

                @pl.when(k + 1 < AGG_FULL_CHUNKS)
                def _():
                    start_gather((b + 1) % 3, (b + 1) % 2,
                                 prior_scatter=(k + 1 >= 2))

                scatter(b % 2)

        wait_scatter(0)
        wait_scatter(1)

        bt = pl.multiple_of(ebase + AGG_FULL_CHUNKS * CHUNK, 8)
        pltpu.sync_copy(ei_hbm.at[0, pl.ds(bt, AGG_TAIL)], srcv_t)
        pltpu.sync_copy(ei_hbm.at[1, pl.ds(bt, AGG_TAIL)], dstv_t)
        idxv_t[...] = srcv_t[...] + row_off
        pltpu.async_copy(h_hbm.at[idxv_t], rowsv_t, se0).wait()
        pltpu.sync_copy(rowsv_t, acc.at[dstv_t], add=True)

        plsc.subcore_barrier()
        pltpu.sync_copy(acc.at[pl.ds(r0, ROWS_PER_TILE)],
                        out_hbm.at[pl.ds(row_off + r0, ROWS_PER_TILE)])

    return agg


_sc_agg128 = _make_sc_agg(128)


HD3 = 64
AGG3_CHUNKS = EDGES_PER_TILE_DEG // CHUNK
AGG3_TAIL = EDGES_PER_TILE_DEG - AGG3_CHUNKS * CHUNK


@functools.partial(
    pl.kernel,
    out_type=jax.ShapeDtypeStruct((NC, NP, HD3), jnp.float32),
    mesh=_mesh(),
    scratch_types=[
        pltpu.VMEM((3, 2, CHUNK), jnp.int32),
        pltpu.VMEM((3, CHUNK), jnp.int32),
        pltpu.VMEM((3, CHUNK), jnp.int32),
        pltpu.VMEM((3, CHUNK, HD3), jnp.float32),
        pltpu.VMEM((AGG3_TAIL,), jnp.int32),
        pltpu.VMEM((AGG3_TAIL,), jnp.int32),
        pltpu.VMEM((AGG3_TAIL, HD3), jnp.float32),
        pltpu.VMEM_SHARED((NP, HD3), jnp.float32),
        pltpu.SemaphoreType.DMA,
        pltpu.SemaphoreType.DMA,
        pltpu.SemaphoreType.DMA,
        pltpu.SemaphoreType.DMA,
        pltpu.SemaphoreType.DMA,
        pltpu.SemaphoreType.DMA,
        pltpu.SemaphoreType.DMA,
        pltpu.SemaphoreType.DMA,
        pltpu.SemaphoreType.DMA,
    ],
    compiler_params=pltpu.CompilerParams(use_tc_tiling_on_sc=False),
)
def _sc_agg64(hg_hbm, hh_hbm, ei_hbm, out_hbm,
              eiv, idxv, dstv, rowsv, srcv_t, dstv_t, rowsv_t,
              acc, se0, se1, se2, sg0, sg1, sg2, ss0, ss1, ss2):
    c = lax.axis_index("c")
    s = lax.axis_index("s")
    r0 = s * ROWS_PER_TILE
    sems_e = (se0, se1, se2)
    sems_g = (sg0, sg1, sg2)
    sems_s = (ss0, ss1, ss2)

    pltpu.sync_copy(hh_hbm.at[pl.ds(r0, ROWS_PER_TILE)],
                    acc.at[pl.ds(r0, ROWS_PER_TILE)])
    plsc.subcore_barrier()

    ebase = c * (E // NC) + s * EDGES_PER_TILE_DEG

    def fetch_ei(k, e):
        b = pl.multiple_of(ebase + k * CHUNK, 8)
        pltpu.async_copy(ei_hbm.at[:, pl.ds(b, CHUNK)], eiv.at[e], sems_e[e])

    def wait_scatter(g):
        pltpu.make_async_copy(rowsv.at[g], acc.at[dstv.at[g]],
                              sems_s[g]).wait()

    def start_gather(e, prior_scatter=None):
        pltpu.make_async_copy(ei_hbm.at[:, pl.ds(0, CHUNK)],
                              eiv.at[e], sems_e[e]).wait()
        if prior_scatter is not None:
            @pl.when(prior_scatter)
            def _():
                wait_scatter(e)
        for i in range(CHUNK // 16):
            sl = pl.ds(i * 16, 16)
            idxv[e, sl] = eiv[e, 0, sl]
            dstv[e, sl] = eiv[e, 1, sl]
        pltpu.async_copy(hg_hbm.at[idxv.at[e]], rowsv.at[e], sems_g[e])

    def scatter(g):
        pltpu.make_async_copy(hg_hbm.at[idxv.at[g]], rowsv.at[g],
                              sems_g[g]).wait()
        pltpu.async_copy(rowsv.at[g], acc.at[dstv.at[g]], sems_s[g], add=True)

    fetch_ei(0, 0)
    fetch_ei(1, 1)
    start_gather(0)

    @pl.loop(0, AGG3_CHUNKS, step=3)
    def _(j):
        for b in range(3):
            k = j + b

            @pl.when(k + 2 < AGG3_CHUNKS)
            def _():
                fetch_ei(k + 2, (b + 2) % 3)

            @pl.when(k + 1 < AGG3_CHUNKS)
            def _():
                start_gather((b + 1) % 3, prior_scatter=(k + 1 >= 3))

            scatter(b % 3)

    wait_scatter(0)
    wait_scatter(1)
    wait_scatter(2)

    bt = pl.multiple_of(ebase + AGG3_CHUNKS * CHUNK, 8)
    pltpu.sync_copy(ei_hbm.at[0, pl.ds(bt, AGG3_TAIL)], srcv_t)
    pltpu.sync_copy(ei_hbm.at[1, pl.ds(bt, AGG3_TAIL)], dstv_t)
    pltpu.async_copy(hg_hbm.at[srcv_t], rowsv_t, se0).wait()
    pltpu.sync_copy(rowsv_t, acc.at[dstv_t], add=True)

    plsc.subcore_barrier()
    pltpu.sync_copy(acc.at[pl.ds(r0, ROWS_PER_TILE)],
                    out_hbm.at[c, pl.ds(r0, ROWS_PER_TILE)])


def _split_store(out_ref, h):
    hw = h.shape[1] // 2
    out_ref[0:N, :] = h[:, 0:hw]
    out_ref[NP:NP + N, :] = h[:, hw:]
    pad = jnp.zeros((NP - N, hw), jnp.float32)
    out_ref[N:NP, :] = pad
    out_ref[NP + N:2 * NP, :] = pad


def _dinv(degp_ref):
    deg = degp_ref[0, 0:N, 0:1] + degp_ref[1, 0:N, 0:1] + 1.0
    return lax.rsqrt(deg)


def _tc1_body(degp_ref, x_ref, w_ref, out_ref):
    dinv = _dinv(degp_ref)
    h = jnp.dot(x_ref[...], w_ref[...], preferred_element_type=jnp.float32)
    h = h * dinv
    _split_store(out_ref, h)


def _tc_mid_body(degp_ref, agg_ref, b_ref, g_ref, be_ref, w_ref, out_ref):
    dinv = _dinv(degp_ref)
    z = jnp.concatenate([agg_ref[0:N, :], agg_ref[NP:NP + N, :]], axis=1)
    z = z * dinv + b_ref[...]
    z = jnp.maximum(z, 0.0)
    mean = jnp.mean(z, axis=0, keepdims=True)
    var = jnp.mean((z - mean) * (z - mean), axis=0, keepdims=True)
    z = g_ref[...] * (z - mean) * lax.rsqrt(var + 1e-5) + be_ref[...]
    h = jnp.dot(z, w_ref[...], preferred_element_type=jnp.float32)
    h = h * dinv
    _split_store(out_ref, h)


def _tc3_body(degp_ref, agg_ref, b_ref, g_ref, be_ref, w_ref,
              outg_ref, outh_ref):
    dinv = _dinv(degp_ref)
    z = jnp.concatenate([agg_ref[0:N, :], agg_ref[NP:NP + N, :]], axis=1)
    z = z * dinv + b_ref[...]
    z = jnp.maximum(z, 0.0)
    mean = jnp.mean(z, axis=0, keepdims=True)
    var = jnp.mean((z - mean) * (z - mean), axis=0, keepdims=True)
    z = g_ref[...] * (z - mean) * lax.rsqrt(var + 1e-5) + be_ref[...]
    h = jnp.dot(z, w_ref[...], preferred_element_type=jnp.float32)
    h = h * dinv
    pad = jnp.zeros((NP - N, HD3), jnp.float32)
    outg_ref[0:N, :] = h
    outg_ref[N:NP, :] = pad
    outh_ref[0:N, :] = 0.5 * h
    outh_ref[N:NP, :] = pad


def _tc_final_body(degp_ref, agg_ref, b_ref, out_ref):
    dinv = _dinv(degp_ref)
    z = agg_ref[0, 0:N, :] + agg_ref[1, 0:N, :]
    z = z * dinv + b_ref[...]
    m = jnp.max(z, axis=1, keepdims=True)
    zm = z - m
    lse = jnp.log(jnp.sum(jnp.exp(zm), axis=1, keepdims=True))
    out_ref[...] = zm - lse


def _tc_call(body, out_shape, *args):
    return pl.pallas_call(
        body, out_shape=jax.ShapeDtypeStruct(out_shape, jnp.float32))(*args)


def kernel(x, edge_index, W1, b1, W2, b2, W3, b3, gamma1, beta1, gamma2, beta2):
    src = edge_index[0]
    dst = edge_index[1]

    ones16 = jnp.ones((CHUNK, 16), jnp.float32)
    zeros16 = jnp.zeros((ROWS_PER_TILE, 16), jnp.float32)
    degp = _sc_degree(dst, ones16, zeros16)
    h1 = _tc_call(_tc1_body, (2 * NP, 128), degp, x, W1)
    a1 = _sc_agg128(h1, edge_index)
    h2 = _tc_call(_tc_mid_body, (2 * NP, 128), degp, a1,
                  b1.reshape(1, -1), gamma1.reshape(1, -1),
                  beta1.reshape(1, -1), W2)
    a2 = _sc_agg128(h2, edge_index)
    h3g, h3h = pl.pallas_call(
        _tc3_body,
        out_shape=[jax.ShapeDtypeStruct((NP, HD3), jnp.float32),
                   jax.ShapeDtypeStruct((NP, HD3), jnp.float32)],
    )(degp, a2, b2.reshape(1, -1), gamma2.reshape(1, -1),
      beta2.reshape(1, -1), W3)
    a3 = _sc_agg64(h3g, h3h, edge_index)
    out = _tc_call(_tc_final_body, (N, 64), degp, a3, b3.reshape(1, -1))
    return out


# speedup vs baseline: 1.0403x; 1.0009x over previous
import functools

import jax
import jax.numpy as jnp
from jax import lax
from jax.experimental import pallas as pl
from jax.experimental.pallas import tpu as pltpu
from jax.experimental.pallas import tpu_sc as plsc

N = 10000
NP = 10240
E = 160000
NC = 2
NS = 16
ROWS_PER_TILE = NP // NS
CHUNK = 128

EDGES_PER_TILE_AGG = E // NS
AGG_FULL_CHUNKS = EDGES_PER_TILE_AGG // CHUNK
AGG_TAIL = EDGES_PER_TILE_AGG - AGG_FULL_CHUNKS * CHUNK

EDGES_PER_TILE_DEG = E // (NC * NS)
DEG_FULL_CHUNKS = EDGES_PER_TILE_DEG // CHUNK
DEG_TAIL = EDGES_PER_TILE_DEG - DEG_FULL_CHUNKS * CHUNK


def _mesh():
    return plsc.VectorSubcoreMesh(core_axis_name="c", subcore_axis_name="s")


@functools.partial(
    pl.kernel,
    out_type=jax.ShapeDtypeStruct((NC, NP, 16), jnp.float32),
    mesh=_mesh(),
    scratch_types=[
        pltpu.VMEM((2, CHUNK), jnp.int32),
        pltpu.VMEM((DEG_TAIL,), jnp.int32),
        pltpu.VMEM((CHUNK, 16), jnp.float32),
        pltpu.VMEM((ROWS_PER_TILE, 16), jnp.float32),
        pltpu.VMEM_SHARED((NP, 16), jnp.float32),
        pltpu.SemaphoreType.DMA,
        pltpu.SemaphoreType.DMA,
    ],
    compiler_params=pltpu.CompilerParams(use_tc_tiling_on_sc=False),
)
def _sc_degree(dst_hbm, ones_hbm, zeros_hbm, out_hbm, dstv, dstv_t, onesv,
               zerov, acc, sd0, sd1):
    c = lax.axis_index("c")
    s = lax.axis_index("s")
    sems = (sd0, sd1)

    pltpu.sync_copy(ones_hbm, onesv)
    pltpu.sync_copy(zeros_hbm, zerov)

    r0 = s * ROWS_PER_TILE
    pltpu.sync_copy(zerov, acc.at[pl.ds(r0, ROWS_PER_TILE)])
    plsc.subcore_barrier()

    base = c * (E // NC) + s * EDGES_PER_TILE_DEG

    def fetch_d(buf, j):
        b = pl.multiple_of(base + j * CHUNK, 8)
        pltpu.async_copy(dst_hbm.at[pl.ds(b, CHUNK)], dstv.at[buf], sems[buf])

    def drain_d(buf):
        pltpu.make_async_copy(dst_hbm.at[pl.ds(0, CHUNK)], dstv.at[buf],
                              sems[buf]).wait()
        pltpu.sync_copy(onesv, acc.at[dstv.at[buf]], add=True)

    fetch_d(0, 0)

    @pl.loop(0, DEG_FULL_CHUNKS - 1, step=2)
    def _(j):
        fetch_d(1, j + 1)
        drain_d(0)
        fetch_d(0, j + 2)
        drain_d(1)

    drain_d(0)

    bt = pl.multiple_of(base + DEG_FULL_CHUNKS * CHUNK, 8)
    pltpu.sync_copy(dst_hbm.at[pl.ds(bt, DEG_TAIL)], dstv_t)
    pltpu.sync_copy(onesv.at[pl.ds(0, DEG_TAIL)], acc.at[dstv_t], add=True)

    plsc.subcore_barrier()
    pltpu.sync_copy(acc.at[pl.ds(r0, ROWS_PER_TILE)],
                    out_hbm.at[c, pl.ds(r0, ROWS_PER_TILE)])


def _make_sc_agg(HD, dt=jnp.float32):
    @functools.partial(
        pl.kernel,
        out_type=jax.ShapeDtypeStruct((2 * NP, HD), dt),
        mesh=_mesh(),
        scratch_types=[
            pltpu.VMEM((3, 2, CHUNK), jnp.int32),
            pltpu.VMEM((2, CHUNK), jnp.int32),
            pltpu.VMEM((2, CHUNK), jnp.int32),
            pltpu.VMEM((2, CHUNK, HD), dt),
            pltpu.VMEM((AGG_TAIL,), jnp.int32),
            pltpu.VMEM((AGG_TAIL,), jnp.int32),
            pltpu.VMEM((AGG_TAIL,), jnp.int32),
            pltpu.VMEM((AGG_TAIL, HD), dt),
            pltpu.VMEM_SHARED((NP, HD), dt),
            pltpu.SemaphoreType.DMA,
            pltpu.SemaphoreType.DMA,
            pltpu.SemaphoreType.DMA,
            pltpu.SemaphoreType.DMA,
            pltpu.SemaphoreType.DMA,
            pltpu.SemaphoreType.DMA,
            pltpu.SemaphoreType.DMA,
        ],
        compiler_params=pltpu.CompilerParams(use_tc_tiling_on_sc=False),
    )
    def agg(h_hbm, ei_hbm, out_hbm,
            eiv, idxv, dstv, rowsv, srcv_t, dstv_t, idxv_t, rowsv_t,
            acc, se0, se1, se2, sg0, sg1, ss0, ss1):
        c = lax.axis_index("c")
        s = lax.axis_index("s")
        r0 = s * ROWS_PER_TILE
        row_off = c * NP
        sems_e = (se0, se1, se2)
        sems_g = (sg0, sg1)
        sems_s = (ss0, ss1)

        pltpu.sync_copy(h_hbm.at[pl.ds(row_off + r0, ROWS_PER_TILE)],
                        acc.at[pl.ds(r0, ROWS_PER_TILE)])
        plsc.subcore_barrier()

        ebase = s * EDGES_PER_TILE_AGG

        def fetch_ei(k, e):
            b = pl.multiple_of(ebase + k * CHUNK, 8)
            pltpu.async_copy(ei_hbm.at[:, pl.ds(b, CHUNK)], eiv.at[e],
                             sems_e[e])

        def wait_scatter(g):
            pltpu.make_async_copy(rowsv.at[g], acc.at[dstv.at[g]],
                                  sems_s[g]).wait()

        def start_gather(e, g, prior_scatter=None):
            pltpu.make_async_copy(ei_hbm.at[:, pl.ds(0, CHUNK)],
                                  eiv.at[e], sems_e[e]).wait()
            if prior_scatter is not None:
                @pl.when(prior_scatter)
                def _():
                    wait_scatter(g)
            for i in range(CHUNK // 16):
                sl = pl.ds(i * 16, 16)
                idxv[g, sl] = eiv[e, 0, sl] + row_off
                dstv[g, sl] = eiv[e, 1, sl]
            pltpu.async_copy(h_hbm.at[idxv.at[g]], rowsv.at[g], sems_g[g])

        def scatter(g):
            pltpu.make_async_copy(h_hbm.at[idxv.at[g]], rowsv.at[g],
                                  sems_g[g]).wait()
            pltpu.async_copy(rowsv.at[g], acc.at[dstv.at[g]], sems_s[g],
                             add=True)

        fetch_ei(0, 0)
        fetch_ei(1, 1)
        start_gather(0, 0)

        @pl.loop(0, AGG_FULL_CHUNKS, step=6)
        def _(j):
            for b in range(6):
                k = j + b

                @pl.when(k + 2 < AGG_FULL_CHUNKS)
                def _():
                    fetch_ei(k + 2, (b + 2) % 3)

                @pl.when(k + 1 < AGG_FULL_CHUNKS)
                def _():
                    start_gather((b + 1) % 3, (b + 1) % 2,
                                 prior_scatter=(k + 1 >= 2))

                scatter(b % 2)

        wait_scatter(0)
        wait_scatter(1)

        bt = pl.multiple_of(ebase + AGG_FULL_CHUNKS * CHUNK, 8)
        pltpu.sync_copy(ei_hbm.at[0, pl.ds(bt, AGG_TAIL)], srcv_t)
        pltpu.sync_copy(ei_hbm.at[1, pl.ds(bt, AGG_TAIL)], dstv_t)
        idxv_t[...] = srcv_t[...] + row_off
        pltpu.async_copy(h_hbm.at[idxv_t], rowsv_t, se0).wait()
        pltpu.sync_copy(rowsv_t, acc.at[dstv_t], add=True)

        plsc.subcore_barrier()
        pltpu.sync_copy(acc.at[pl.ds(r0, ROWS_PER_TILE)],
                        out_hbm.at[pl.ds(row_off + r0, ROWS_PER_TILE)])

    return agg


_sc_agg128 = _make_sc_agg(128, jnp.bfloat16)


HD3 = 64
AGG3_CHUNKS = EDGES_PER_TILE_DEG // CHUNK
AGG3_TAIL = EDGES_PER_TILE_DEG - AGG3_CHUNKS * CHUNK


@functools.partial(
    pl.kernel,
    out_type=jax.ShapeDtypeStruct((NC, NP, HD3), jnp.float32),
    mesh=_mesh(),
    scratch_types=[
        pltpu.VMEM((3, 2, CHUNK), jnp.int32),
        pltpu.VMEM((3, CHUNK), jnp.int32),
        pltpu.VMEM((3, CHUNK), jnp.int32),
        pltpu.VMEM((3, CHUNK, HD3), jnp.float32),
        pltpu.VMEM((AGG3_TAIL,), jnp.int32),
        pltpu.VMEM((AGG3_TAIL,), jnp.int32),
        pltpu.VMEM((AGG3_TAIL, HD3), jnp.float32),
        pltpu.VMEM_SHARED((NP, HD3), jnp.float32),
        pltpu.SemaphoreType.DMA,
        pltpu.SemaphoreType.DMA,
        pltpu.SemaphoreType.DMA,
        pltpu.SemaphoreType.DMA,
        pltpu.SemaphoreType.DMA,
        pltpu.SemaphoreType.DMA,
        pltpu.SemaphoreType.DMA,
        pltpu.SemaphoreType.DMA,
        pltpu.SemaphoreType.DMA,
    ],
    compiler_params=pltpu.CompilerParams(use_tc_tiling_on_sc=False),
)
def _sc_agg64(hg_hbm, hh_hbm, ei_hbm, out_hbm,
              eiv, idxv, dstv, rowsv, srcv_t, dstv_t, rowsv_t,
              acc, se0, se1, se2, sg0, sg1, sg2, ss0, ss1, ss2):
    c = lax.axis_index("c")
    s = lax.axis_index("s")
    r0 = s * ROWS_PER_TILE
    sems_e = (se0, se1, se2)
    sems_g = (sg0, sg1, sg2)
    sems_s = (ss0, ss1, ss2)

    pltpu.sync_copy(hh_hbm.at[pl.ds(r0, ROWS_PER_TILE)],
                    acc.at[pl.ds(r0, ROWS_PER_TILE)])
    plsc.subcore_barrier()

    ebase = c * (E // NC) + s * EDGES_PER_TILE_DEG

    def fetch_ei(k, e):
        b = pl.multiple_of(ebase + k * CHUNK, 8)
        pltpu.async_copy(ei_hbm.at[:, pl.ds(b, CHUNK)], eiv.at[e], sems_e[e])

    def wait_scatter(g):
        pltpu.make_async_copy(rowsv.at[g], acc.at[dstv.at[g]],
                              sems_s[g]).wait()

    def start_gather(e, prior_scatter=None):
        pltpu.make_async_copy(ei_hbm.at[:, pl.ds(0, CHUNK)],
                              eiv.at[e], sems_e[e]).wait()
        if prior_scatter is not None:
            @pl.when(prior_scatter)
            def _():
                wait_scatter(e)
        for i in range(CHUNK // 16):
            sl = pl.ds(i * 16, 16)
            idxv[e, sl] = eiv[e, 0, sl]
            dstv[e, sl] = eiv[e, 1, sl]
        pltpu.async_copy(hg_hbm.at[idxv.at[e]], rowsv.at[e], sems_g[e])

    def scatter(g):
        pltpu.make_async_copy(hg_hbm.at[idxv.at[g]], rowsv.at[g],
                              sems_g[g]).wait()
        pltpu.async_copy(rowsv.at[g], acc.at[dstv.at[g]], sems_s[g], add=True)

    fetch_ei(0, 0)
    fetch_ei(1, 1)
    start_gather(0)

    @pl.loop(0, AGG3_CHUNKS, step=3)
    def _(j):
        for b in range(3):
            k = j + b

            @pl.when(k + 2 < AGG3_CHUNKS)
            def _():
                fetch_ei(k + 2, (b + 2) % 3)

            @pl.when(k + 1 < AGG3_CHUNKS)
            def _():
                start_gather((b + 1) % 3, prior_scatter=(k + 1 >= 3))

            scatter(b % 3)

    wait_scatter(0)
    wait_scatter(1)
    wait_scatter(2)

    bt = pl.multiple_of(ebase + AGG3_CHUNKS * CHUNK, 8)
    pltpu.sync_copy(ei_hbm.at[0, pl.ds(bt, AGG3_TAIL)], srcv_t)
    pltpu.sync_copy(ei_hbm.at[1, pl.ds(bt, AGG3_TAIL)], dstv_t)
    pltpu.async_copy(hg_hbm.at[srcv_t], rowsv_t, se0).wait()
    pltpu.sync_copy(rowsv_t, acc.at[dstv_t], add=True)

    plsc.subcore_barrier()
    pltpu.sync_copy(acc.at[pl.ds(r0, ROWS_PER_TILE)],
                    out_hbm.at[c, pl.ds(r0, ROWS_PER_TILE)])


def _split_store(out_ref, h):
    h = h.astype(out_ref.dtype)
    hw = h.shape[1] // 2
    out_ref[0:N, :] = h[:, 0:hw]
    out_ref[NP:NP + N, :] = h[:, hw:]
    pad = jnp.zeros((NP - N, hw), out_ref.dtype)
    out_ref[N:NP, :] = pad
    out_ref[NP + N:2 * NP, :] = pad


def _dinv(degp_ref):
    deg = degp_ref[0, 0:N, 0:1] + degp_ref[1, 0:N, 0:1] + 1.0
    return lax.rsqrt(deg)


def _tc1_body(degp_ref, x_ref, w_ref, out_ref):
    dinv = _dinv(degp_ref)
    h = jnp.dot(x_ref[...], w_ref[...], preferred_element_type=jnp.float32)
    h = h * dinv
    _split_store(out_ref, h)


def _tc_mid_body(degp_ref, agg_ref, b_ref, g_ref, be_ref, w_ref, out_ref):
    dinv = _dinv(degp_ref)
    z = jnp.concatenate([agg_ref[0:N, :], agg_ref[NP:NP + N, :]], axis=1)
    z = z.astype(jnp.float32) * dinv + b_ref[...]
    z = jnp.maximum(z, 0.0)
    mean = jnp.mean(z, axis=0, keepdims=True)
    var = jnp.mean((z - mean) * (z - mean), axis=0, keepdims=True)
    z = g_ref[...] * (z - mean) * lax.rsqrt(var + 1e-5) + be_ref[...]
    h = jnp.dot(z, w_ref[...], preferred_element_type=jnp.float32)
    h = h * dinv
    _split_store(out_ref, h)


def _tc3_body(degp_ref, agg_ref, b_ref, g_ref, be_ref, w_ref,
              outg_ref, outh_ref):
    dinv = _dinv(degp_ref)
    z = jnp.concatenate([agg_ref[0:N, :], agg_ref[NP:NP + N, :]], axis=1)
    z = z.astype(jnp.float32) * dinv + b_ref[...]
    z = jnp.maximum(z, 0.0)
    mean = jnp.mean(z, axis=0, keepdims=True)
    var = jnp.mean((z - mean) * (z - mean), axis=0, keepdims=True)
    z = g_ref[...] * (z - mean) * lax.rsqrt(var + 1e-5) + be_ref[...]
    h = jnp.dot(z, w_ref[...], preferred_element_type=jnp.float32)
    h = h * dinv
    pad = jnp.zeros((NP - N, HD3), jnp.float32)
    outg_ref[0:N, :] = h
    outg_ref[N:NP, :] = pad
    outh_ref[0:N, :] = 0.5 * h
    outh_ref[N:NP, :] = pad


def _tc_final_body(degp_ref, agg_ref, b_ref, out_ref):
    dinv = _dinv(degp_ref)
    z = agg_ref[0, 0:N, :] + agg_ref[1, 0:N, :]
    z = z * dinv + b_ref[...]
    m = jnp.max(z, axis=1, keepdims=True)
    zm = z - m
    lse = jnp.log(jnp.sum(jnp.exp(zm), axis=1, keepdims=True))
    out_ref[...] = zm - lse


def _tc_call(body, out_shape, *args, dtype=jnp.float32):
    return pl.pallas_call(
        body, out_shape=jax.ShapeDtypeStruct(out_shape, dtype))(*args)


def kernel(x, edge_index, W1, b1, W2, b2, W3, b3, gamma1, beta1, gamma2, beta2):
    src = edge_index[0]
    dst = edge_index[1]

    ones16 = jnp.ones((CHUNK, 16), jnp.float32)
    zeros16 = jnp.zeros((ROWS_PER_TILE, 16), jnp.float32)
    degp = _sc_degree(dst, ones16, zeros16)
    h1 = _tc_call(_tc1_body, (2 * NP, 128), degp, x, W1,
                  dtype=jnp.bfloat16)
    a1 = _sc_agg128(h1, edge_index)
    h2 = _tc_call(_tc_mid_body, (2 * NP, 128), degp, a1,
                  b1.reshape(1, -1), gamma1.reshape(1, -1),
                  beta1.reshape(1, -1), W2, dtype=jnp.bfloat16)
    a2 = _sc_agg128(h2, edge_index)
    h3g, h3h = pl.pallas_call(
        _tc3_body,
        out_shape=[jax.ShapeDtypeStruct((NP, HD3), jnp.float32),
                   jax.ShapeDtypeStruct((NP, HD3), jnp.float32)],
    )(degp, a2, b2.reshape(1, -1), gamma2.reshape(1, -1),
      beta2.reshape(1, -1), W3)
    a3 = _sc_agg64(h3g, h3h, edge_index)
    out = _tc_call(_tc_final_body, (N, 64), degp, a3, b3.reshape(1, -1))
    return out


# speedup vs baseline: 1.0486x; 1.0080x over previous
import functools

import jax
import jax.numpy as jnp
from jax import lax
from jax.experimental import pallas as pl
from jax.experimental.pallas import tpu as pltpu
from jax.experimental.pallas import tpu_sc as plsc

N = 10000
NP = 10240
E = 160000
NC = 2
NS = 16
ROWS_PER_TILE = NP // NS
CHUNK = 128

EDGES_PER_TILE_AGG = E // NS
AGG_FULL_CHUNKS = EDGES_PER_TILE_AGG // CHUNK
AGG_TAIL = EDGES_PER_TILE_AGG - AGG_FULL_CHUNKS * CHUNK

EDGES_PER_TILE_DEG = E // (NC * NS)
DEG_FULL_CHUNKS = EDGES_PER_TILE_DEG // CHUNK
DEG_TAIL = EDGES_PER_TILE_DEG - DEG_FULL_CHUNKS * CHUNK


def _mesh():
    return plsc.VectorSubcoreMesh(core_axis_name="c", subcore_axis_name="s")


@functools.partial(
    pl.kernel,
    out_type=jax.ShapeDtypeStruct((NC, NP, 16), jnp.float32),
    mesh=_mesh(),
    scratch_types=[
        pltpu.VMEM((2, CHUNK), jnp.int32),
        pltpu.VMEM((DEG_TAIL,), jnp.int32),
        pltpu.VMEM((CHUNK, 16), jnp.float32),
        pltpu.VMEM((ROWS_PER_TILE, 16), jnp.float32),
        pltpu.VMEM_SHARED((NP, 16), jnp.float32),
        pltpu.SemaphoreType.DMA,
        pltpu.SemaphoreType.DMA,
    ],
    compiler_params=pltpu.CompilerParams(use_tc_tiling_on_sc=False),
)
def _sc_degree(dst_hbm, ones_hbm, zeros_hbm, out_hbm, dstv, dstv_t, onesv,
               zerov, acc, sd0, sd1):
    c = lax.axis_index("c")
    s = lax.axis_index("s")
    sems = (sd0, sd1)

    pltpu.sync_copy(ones_hbm, onesv)
    pltpu.sync_copy(zeros_hbm, zerov)

    r0 = s * ROWS_PER_TILE
    pltpu.sync_copy(zerov, acc.at[pl.ds(r0, ROWS_PER_TILE)])
    plsc.subcore_barrier()

    base = c * (E // NC) + s * EDGES_PER_TILE_DEG

    def fetch_d(buf, j):
        b = pl.multiple_of(base + j * CHUNK, 8)
        pltpu.async_copy(dst_hbm.at[pl.ds(b, CHUNK)], dstv.at[buf], sems[buf])

    def drain_d(buf):
        pltpu.make_async_copy(dst_hbm.at[pl.ds(0, CHUNK)], dstv.at[buf],
                              sems[buf]).wait()
        pltpu.sync_copy(onesv, acc.at[dstv.at[buf]], add=True)

    fetch_d(0, 0)

    @pl.loop(0, DEG_FULL_CHUNKS - 1, step=2)
    def _(j):
        fetch_d(1, j + 1)
        drain_d(0)
        fetch_d(0, j + 2)
        drain_d(1)

    drain_d(0)

    bt = pl.multiple_of(base + DEG_FULL_CHUNKS * CHUNK, 8)
    pltpu.sync_copy(dst_hbm.at[pl.ds(bt, DEG_TAIL)], dstv_t)
    pltpu.sync_copy(onesv.at[pl.ds(0, DEG_TAIL)], acc.at[dstv_t], add=True)

    plsc.subcore_barrier()
    pltpu.sync_copy(acc.at[pl.ds(r0, ROWS_PER_TILE)],
                    out_hbm.at[c, pl.ds(r0, ROWS_PER_TILE)])


def _make_sc_agg(HD, dt=jnp.float32):
    @functools.partial(
        pl.kernel,
        out_type=jax.ShapeDtypeStruct((2 * NP, HD), dt),
        mesh=_mesh(),
        scratch_types=[
            pltpu.VMEM((3, 2, CHUNK), jnp.int32),
            pltpu.VMEM((2, CHUNK), jnp.int32),
            pltpu.VMEM((2, CHUNK), jnp.int32),
            pltpu.VMEM((2, CHUNK, HD), dt),
            pltpu.VMEM((AGG_TAIL,), jnp.int32),
            pltpu.VMEM((AGG_TAIL,), jnp.int32),
            pltpu.VMEM((AGG_TAIL,), jnp.int32),
            pltpu.VMEM((AGG_TAIL, HD), dt),
            pltpu.VMEM_SHARED((NP, HD), dt),
            pltpu.SemaphoreType.DMA,
            pltpu.SemaphoreType.DMA,
            pltpu.SemaphoreType.DMA,
            pltpu.SemaphoreType.DMA,
            pltpu.SemaphoreType.DMA,
            pltpu.SemaphoreType.DMA,
            pltpu.SemaphoreType.DMA,
        ],
        compiler_params=pltpu.CompilerParams(use_tc_tiling_on_sc=False),
    )
    def agg(h_hbm, ei_hbm, out_hbm,
            eiv, idxv, dstv, rowsv, srcv_t, dstv_t, idxv_t, rowsv_t,
            acc, se0, se1, se2, sg0, sg1, ss0, ss1):
        c = lax.axis_index("c")
        s = lax.axis_index("s")
        r0 = s * ROWS_PER_TILE
        row_off = c * NP
        sems_e = (se0, se1, se2)
        sems_g = (sg0, sg1)
        sems_s = (ss0, ss1)

        ebase = s * EDGES_PER_TILE_AGG

        def fetch_ei(k, e):
            b = pl.multiple_of(ebase + k * CHUNK, 8)
            pltpu.async_copy(ei_hbm.at[:, pl.ds(b, CHUNK)], eiv.at[e],
                             sems_e[e])

        def wait_scatter(g):
            pltpu.make_async_copy(rowsv.at[g], acc.at[dstv.at[g]],
                                  sems_s[g]).wait()

        def start_gather(e, g, prior_scatter=None):
            pltpu.make_async_copy(ei_hbm.at[:, pl.ds(0, CHUNK)],
                                  eiv.at[e], sems_e[e]).wait()
            if prior_scatter is not None:
                @pl.when(prior_scatter)
                def _():
                    wait_scatter(g)
            for i in range(CHUNK // 16):
                sl = pl.ds(i * 16, 16)
                idxv[g, sl] = eiv[e, 0, sl] + row_off
                dstv[g, sl] = eiv[e, 1, sl]
            pltpu.async_copy(h_hbm.at[idxv.at[g]], rowsv.at[g], sems_g[g])

        def scatter(g):
            pltpu.make_async_copy(h_hbm.at[idxv.at[g]], rowsv.at[g],
                                  sems_g[g]).wait()
            pltpu.async_copy(rowsv.at[g], acc.at[dstv.at[g]], sems_s[g],
                             add=True)

        fetch_ei(0, 0)
        fetch_ei(1, 1)
        start_gather(0, 0)

        pltpu.sync_copy(h_hbm.at[pl.ds(row_off + r0, ROWS_PER_TILE)],
                        acc.at[pl.ds(r0, ROWS_PER_TILE)])
        plsc.subcore_barrier()

        @pl.loop(0, AGG_FULL_CHUNKS, step=6)
        def _(j):
            for b in range(6):
                k = j + b

                @pl.when(k + 2 < AGG_FULL_CHUNKS)
                def _():
                    fetch_ei(k + 2, (b + 2) % 3)

                @pl.when(k + 1 < AGG_FULL_CHUNKS)
                def _():
                    start_gather((b + 1) % 3, (b + 1) % 2,
                                 prior_scatter=(k + 1 >= 2))

                scatter(b % 2)

        wait_scatter(0)
        wait_scatter(1)

        bt = pl.multiple_of(ebase + AGG_FULL_CHUNKS * CHUNK, 8)
        pltpu.sync_copy(ei_hbm.at[0, pl.ds(bt, AGG_TAIL)], srcv_t)
        pltpu.sync_copy(ei_hbm.at[1, pl.ds(bt, AGG_TAIL)], dstv_t)
        idxv_t[...] = srcv_t[...] + row_off
        pltpu.async_copy(h_hbm.at[idxv_t], rowsv_t, se0).wait()
        pltpu.sync_copy(rowsv_t, acc.at[dstv_t], add=True)

        plsc.subcore_barrier()
        pltpu.sync_copy(acc.at[pl.ds(r0, ROWS_PER_TILE)],
                        out_hbm.at[pl.ds(row_off + r0, ROWS_PER_TILE)])

    return agg


_sc_agg128 = _make_sc_agg(128)


HD3 = 64
AGG3_CHUNKS = EDGES_PER_TILE_DEG // CHUNK
AGG3_TAIL = EDGES_PER_TILE_DEG - AGG3_CHUNKS * CHUNK


@functools.partial(
    pl.kernel,
    out_type=jax.ShapeDtypeStruct((NC, NP, HD3), jnp.float32),
    mesh=_mesh(),
    scratch_types=[
        pltpu.VMEM((3, 2, CHUNK), jnp.int32),
        pltpu.VMEM((3, CHUNK), jnp.int32),
        pltpu.VMEM((3, CHUNK), jnp.int32),
        pltpu.VMEM((3, CHUNK, HD3), jnp.float32),
        pltpu.VMEM((AGG3_TAIL,), jnp.int32),
        pltpu.VMEM((AGG3_TAIL,), jnp.int32),
        pltpu.VMEM((AGG3_TAIL, HD3), jnp.float32),
        pltpu.VMEM_SHARED((NP, HD3), jnp.float32),
        pltpu.SemaphoreType.DMA,
        pltpu.SemaphoreType.DMA,
        pltpu.SemaphoreType.DMA,
        pltpu.SemaphoreType.DMA,
        pltpu.SemaphoreType.DMA,
        pltpu.SemaphoreType.DMA,
        pltpu.SemaphoreType.DMA,
        pltpu.SemaphoreType.DMA,
        pltpu.SemaphoreType.DMA,
    ],
    compiler_params=pltpu.CompilerParams(use_tc_tiling_on_sc=False),
)
def _sc_agg64(hg_hbm, hh_hbm, ei_hbm, out_hbm,
              eiv, idxv, dstv, rowsv, srcv_t, dstv_t, rowsv_t,
              acc, se0, se1, se2, sg0, sg1, sg2, ss0, ss1, ss2):
    c = lax.axis_index("c")
    s = lax.axis_index("s")
    r0 = s * ROWS_PER_TILE
    sems_e = (se0, se1, se2)
    sems_g = (sg0, sg1, sg2)
    sems_s = (ss0, ss1, ss2)

    ebase = c * (E // NC) + s * EDGES_PER_TILE_DEG

    def fetch_ei(k, e):
        b = pl.multiple_of(ebase + k * CHUNK, 8)
        pltpu.async_copy(ei_hbm.at[:, pl.ds(b, CHUNK)], eiv.at[e], sems_e[e])

    def wait_scatter(g):
        pltpu.make_async_copy(rowsv.at[g], acc.at[dstv.at[g]],
                              sems_s[g]).wait()

    def start_gather(e, prior_scatter=None):
        pltpu.make_async_copy(ei_hbm.at[:, pl.ds(0, CHUNK)],
                              eiv.at[e], sems_e[e]).wait()
        if prior_scatter is not None:
            @pl.when(prior_scatter)
            def _():
                wait_scatter(e)
        for i in range(CHUNK // 16):
            sl = pl.ds(i * 16, 16)
            idxv[e, sl] = eiv[e, 0, sl]
            dstv[e, sl] = eiv[e, 1, sl]
        pltpu.async_copy(hg_hbm.at[idxv.at[e]], rowsv.at[e], sems_g[e])

    def scatter(g):
        pltpu.make_async_copy(hg_hbm.at[idxv.at[g]], rowsv.at[g],
                              sems_g[g]).wait()
        pltpu.async_copy(rowsv.at[g], acc.at[dstv.at[g]], sems_s[g], add=True)

    fetch_ei(0, 0)
    fetch_ei(1, 1)
    start_gather(0)

    pltpu.sync_copy(hh_hbm.at[pl.ds(r0, ROWS_PER_TILE)],
                    acc.at[pl.ds(r0, ROWS_PER_TILE)])
    plsc.subcore_barrier()

    @pl.loop(0, AGG3_CHUNKS, step=3)
    def _(j):
        for b in range(3):
            k = j + b

            @pl.when(k + 2 < AGG3_CHUNKS)
            def _():
                fetch_ei(k + 2, (b + 2) % 3)

            @pl.when(k + 1 < AGG3_CHUNKS)
            def _():
                start_gather((b + 1) % 3, prior_scatter=(k + 1 >= 3))

            scatter(b % 3)

    wait_scatter(0)
    wait_scatter(1)
    wait_scatter(2)

    bt = pl.multiple_of(ebase + AGG3_CHUNKS * CHUNK, 8)
    pltpu.sync_copy(ei_hbm.at[0, pl.ds(bt, AGG3_TAIL)], srcv_t)
    pltpu.sync_copy(ei_hbm.at[1, pl.ds(bt, AGG3_TAIL)], dstv_t)
    pltpu.async_copy(hg_hbm.at[srcv_t], rowsv_t, se0).wait()
    pltpu.sync_copy(rowsv_t, acc.at[dstv_t], add=True)

    plsc.subcore_barrier()
    pltpu.sync_copy(acc.at[pl.ds(r0, ROWS_PER_TILE)],
                    out_hbm.at[c, pl.ds(r0, ROWS_PER_TILE)])


def _split_store(out_ref, h):
    h = h.astype(out_ref.dtype)
    hw = h.shape[1] // 2
    out_ref[0:N, :] = h[:, 0:hw]
    out_ref[NP:NP + N, :] = h[:, hw:]
    pad = jnp.zeros((NP - N, hw), out_ref.dtype)
    out_ref[N:NP, :] = pad
    out_ref[NP + N:2 * NP, :] = pad


def _dinv(degp_ref):
    deg = degp_ref[0, 0:N, 0:1] + degp_ref[1, 0:N, 0:1] + 1.0
    return lax.rsqrt(deg)


def _tc1_body(degp_ref, x_ref, w_ref, out_ref):
    dinv = _dinv(degp_ref)
    h = jnp.dot(x_ref[...], w_ref[...], preferred_element_type=jnp.float32)
    h = h * dinv
    _split_store(out_ref, h)


def _tc_mid_body(degp_ref, agg_ref, b_ref, g_ref, be_ref, w_ref, out_ref):
    dinv = _dinv(degp_ref)
    z = jnp.concatenate([agg_ref[0:N, :], agg_ref[NP:NP + N, :]], axis=1)
    z = z.astype(jnp.float32) * dinv + b_ref[...]
    z = jnp.maximum(z, 0.0)
    mean = jnp.mean(z, axis=0, keepdims=True)
    var = jnp.mean((z - mean) * (z - mean), axis=0, keepdims=True)
    z = g_ref[...] * (z - mean) * lax.rsqrt(var + 1e-5) + be_ref[...]
    h = jnp.dot(z, w_ref[...], preferred_element_type=jnp.float32)
    h = h * dinv
    _split_store(out_ref, h)


def _tc3_body(degp_ref, agg_ref, b_ref, g_ref, be_ref, w_ref,
              outg_ref, outh_ref):
    dinv = _dinv(degp_ref)
    z = jnp.concatenate([agg_ref[0:N, :], agg_ref[NP:NP + N, :]], axis=1)
    z = z.astype(jnp.float32) * dinv + b_ref[...]
    z = jnp.maximum(z, 0.0)
    mean = jnp.mean(z, axis=0, keepdims=True)
    var = jnp.mean((z - mean) * (z - mean), axis=0, keepdims=True)
    z = g_ref[...] * (z - mean) * lax.rsqrt(var + 1e-5) + be_ref[...]
    h = jnp.dot(z, w_ref[...], preferred_element_type=jnp.float32)
    h = h * dinv
    pad = jnp.zeros((NP - N, HD3), jnp.float32)
    outg_ref[0:N, :] = h
    outg_ref[N:NP, :] = pad
    outh_ref[0:N, :] = 0.5 * h
    outh_ref[N:NP, :] = pad


def _tc_final_body(degp_ref, agg_ref, b_ref, out_ref):
    dinv = _dinv(degp_ref)
    z = agg_ref[0, 0:N, :] + agg_ref[1, 0:N, :]
    z = z * dinv + b_ref[...]
    m = jnp.max(z, axis=1, keepdims=True)
    zm = z - m
    lse = jnp.log(jnp.sum(jnp.exp(zm), axis=1, keepdims=True))
    out_ref[...] = zm - lse


def _tc_call(body, out_shape, *args, dtype=jnp.float32):
    return pl.pallas_call(
        body, out_shape=jax.ShapeDtypeStruct(out_shape, dtype))(*args)


def kernel(x, edge_index, W1, b1, W2, b2, W3, b3, gamma1, beta1, gamma2, beta2):
    src = edge_index[0]
    dst = edge_index[1]

    ones16 = jnp.ones((CHUNK, 16), jnp.float32)
    zeros16 = jnp.zeros((ROWS_PER_TILE, 16), jnp.float32)
    degp = _sc_degree(dst, ones16, zeros16)
    h1 = _tc_call(_tc1_body, (2 * NP, 128), degp, x, W1)
    a1 = _sc_agg128(h1, edge_index)
    h2 = _tc_call(_tc_mid_body, (2 * NP, 128), degp, a1,
                  b1.reshape(1, -1), gamma1.reshape(1, -1),
                  beta1.reshape(1, -1), W2)
    a2 = _sc_agg128(h2, edge_index)
    h3g, h3h = pl.pallas_call(
        _tc3_body,
        out_shape=[jax.ShapeDtypeStruct((NP, HD3), jnp.float32),
                   jax.ShapeDtypeStruct((NP, HD3), jnp.float32)],
    )(degp, a2, b2.reshape(1, -1), gamma2.reshape(1, -1),
      beta2.reshape(1, -1), W3)
    a3 = _sc_agg64(h3g, h3h, edge_index)
    out = _tc_call(_tc_final_body, (N, 64), degp, a3, b3.reshape(1, -1))
    return out


# speedup vs baseline: 1.0509x; 1.0022x over previous
import functools

import jax
import jax.numpy as jnp
from jax import lax
from jax.experimental import pallas as pl
from jax.experimental.pallas import tpu as pltpu
from jax.experimental.pallas import tpu_sc as plsc

N = 10000
NP = 10240
E = 160000
NC = 2
NS = 16
ROWS_PER_TILE = NP // NS
CHUNK = 128

EDGES_PER_TILE_AGG = E // NS
AGG_FULL_CHUNKS = EDGES_PER_TILE_AGG // CHUNK
AGG_TAIL = EDGES_PER_TILE_AGG - AGG_FULL_CHUNKS * CHUNK

EDGES_PER_TILE_DEG = E // (NC * NS)
DEG_FULL_CHUNKS = EDGES_PER_TILE_DEG // CHUNK
DEG_TAIL = EDGES_PER_TILE_DEG - DEG_FULL_CHUNKS * CHUNK


def _mesh():
    return plsc.VectorSubcoreMesh(core_axis_name="c", subcore_axis_name="s")


@functools.partial(
    pl.kernel,
    out_type=jax.ShapeDtypeStruct((NC, NP, 16), jnp.float32),
    mesh=_mesh(),
    scratch_types=[
        pltpu.VMEM((2, CHUNK), jnp.int32),
        pltpu.VMEM((DEG_TAIL,), jnp.int32),
        pltpu.VMEM((CHUNK, 16), jnp.float32),
        pltpu.VMEM((ROWS_PER_TILE, 16), jnp.float32),
        pltpu.VMEM_SHARED((NP, 16), jnp.float32),
        pltpu.SemaphoreType.DMA,
        pltpu.SemaphoreType.DMA,
    ],
    compiler_params=pltpu.CompilerParams(use_tc_tiling_on_sc=False),
)
def _sc_degree(dst_hbm, ones_hbm, zeros_hbm, out_hbm, dstv, dstv_t, onesv,
               zerov, acc, sd0, sd1):
    c = lax.axis_index("c")
    s = lax.axis_index("s")
    sems = (sd0, sd1)

    base = c * (E // NC) + s * EDGES_PER_TILE_DEG

    def fetch_d(buf, j):
        b = pl.multiple_of(base + j * CHUNK, 8)
        pltpu.async_copy(dst_hbm.at[pl.ds(b, CHUNK)], dstv.at[buf], sems[buf])

    def drain_d(buf):
        pltpu.make_async_copy(dst_hbm.at[pl.ds(0, CHUNK)], dstv.at[buf],
                              sems[buf]).wait()
        pltpu.sync_copy(onesv, acc.at[dstv.at[buf]], add=True)

    fetch_d(0, 0)

    pltpu.sync_copy(ones_hbm, onesv)
    pltpu.sync_copy(zeros_hbm, zerov)

    r0 = s * ROWS_PER_TILE
    pltpu.sync_copy(zerov, acc.at[pl.ds(r0, ROWS_PER_TILE)])
    plsc.subcore_barrier()

    @pl.loop(0, DEG_FULL_CHUNKS - 1, step=2)
    def _(j):
        fetch_d(1, j + 1)
        drain_d(0)
        fetch_d(0, j + 2)
        drain_d(1)

    drain_d(0)

    bt = pl.multiple_of(base + DEG_FULL_CHUNKS * CHUNK, 8)
    pltpu.sync_copy(dst_hbm.at[pl.ds(bt, DEG_TAIL)], dstv_t)
    pltpu.sync_copy(onesv.at[pl.ds(0, DEG_TAIL)], acc.at[dstv_t], add=True)

    plsc.subcore_barrier()
    pltpu.sync_copy(acc.at[pl.ds(r0, ROWS_PER_TILE)],
                    out_hbm.at[c, pl.ds(r0, ROWS_PER_TILE)])


def _make_sc_agg(HD, dt=jnp.float32):
    @functools.partial(
        pl.kernel,
        out_type=jax.ShapeDtypeStruct((2 * NP, HD), dt),
        mesh=_mesh(),
        scratch_types=[
            pltpu.VMEM((3, 2, CHUNK), jnp.int32),
            pltpu.VMEM((2, CHUNK), jnp.int32),
            pltpu.VMEM((2, CHUNK), jnp.int32),
            pltpu.VMEM((2, CHUNK, HD), dt),
            pltpu.VMEM((AGG_TAIL,), jnp.int32),
            pltpu.VMEM((AGG_TAIL,), jnp.int32),
            pltpu.VMEM((AGG_TAIL,), jnp.int32),
            pltpu.VMEM((AGG_TAIL, HD), dt),
            pltpu.VMEM_SHARED((NP, HD), dt),
            pltpu.SemaphoreType.DMA,
            pltpu.SemaphoreType.DMA,
            pltpu.SemaphoreType.DMA,
            pltpu.SemaphoreType.DMA,
            pltpu.SemaphoreType.DMA,
            pltpu.SemaphoreType.DMA,
            pltpu.SemaphoreType.DMA,
        ],
        compiler_params=pltpu.CompilerParams(use_tc_tiling_on_sc=False),
    )
    def agg(h_hbm, ei_hbm, out_hbm,
            eiv, idxv, dstv, rowsv, srcv_t, dstv_t, idxv_t, rowsv_t,
            acc, se0, se1, se2, sg0, sg1, ss0, ss1):
        c = lax.axis_index("c")
        s = lax.axis_index("s")
        r0 = s * ROWS_PER_TILE
        row_off = c * NP
        sems_e = (se0, se1, se2)
        sems_g = (sg0, sg1)
        sems_s = (ss0, ss1)

        ebase = s * EDGES_PER_TILE_AGG

        def fetch_ei(k, e):
            b = pl.multiple_of(ebase + k * CHUNK, 8)
            pltpu.async_copy(ei_hbm.at[:, pl.ds(b, CHUNK)], eiv.at[e],
                             sems_e[e])

        def wait_scatter(g):
            pltpu.make_async_copy(rowsv.at[g], acc.at[dstv.at[g]],
                                  sems_s[g]).wait()

        def start_gather(e, g, prior_scatter=None):
            pltpu.make_async_copy(ei_hbm.at[:, pl.ds(0, CHUNK)],
                                  eiv.at[e], sems_e[e]).wait()
            if prior_scatter is not None:
                @pl.when(prior_scatter)
                def _():
                    wait_scatter(g)
            for i in range(CHUNK // 16):
                sl = pl.ds(i * 16, 16)
                idxv[g, sl] = eiv[e, 0, sl] + row_off
                dstv[g, sl] = eiv[e, 1, sl]
            pltpu.async_copy(h_hbm.at[idxv.at[g]], rowsv.at[g], sems_g[g])

        def scatter(g):
            pltpu.make_async_copy(h_hbm.at[idxv.at[g]], rowsv.at[g],
                                  sems_g[g]).wait()
            pltpu.async_copy(rowsv.at[g], acc.at[dstv.at[g]], sems_s[g],
                             add=True)

        fetch_ei(0, 0)
        fetch_ei(1, 1)
        start_gather(0, 0)

        pltpu.sync_copy(h_hbm.at[pl.ds(row_off + r0, ROWS_PER_TILE)],
                        acc.at[pl.ds(r0, ROWS_PER_TILE)])
        plsc.subcore_barrier()

        @pl.loop(0, AGG_FULL_CHUNKS, step=6)
        def _(j):
            for b in range(6):
                k = j + b

                @pl.when(k + 2 < AGG_FULL_CHUNKS)
                def _():
                    fetch_ei(k + 2, (b + 2) % 3)

                @pl.when(k + 1 < AGG_FULL_CHUNKS)
                def _():
                    start_gather((b + 1) % 3, (b + 1) % 2,
                                 prior_scatter=(k + 1 >= 2))

                scatter(b % 2)

        wait_scatter(0)
        wait_scatter(1)

        bt = pl.multiple_of(ebase + AGG_FULL_CHUNKS * CHUNK, 8)
        pltpu.sync_copy(ei_hbm.at[0, pl.ds(bt, AGG_TAIL)], srcv_t)
        pltpu.sync_copy(ei_hbm.at[1, pl.ds(bt, AGG_TAIL)], dstv_t)
        idxv_t[...] = srcv_t[...] + row_off
        pltpu.async_copy(h_hbm.at[idxv_t], rowsv_t, se0).wait()
        pltpu.sync_copy(rowsv_t, acc.at[dstv_t], add=True)

        plsc.subcore_barrier()
        pltpu.sync_copy(acc.at[pl.ds(r0, ROWS_PER_TILE)],
                        out_hbm.at[pl.ds(row_off + r0, ROWS_PER_TILE)])

    return agg


_sc_agg128 = _make_sc_agg(128)


HD3 = 64
AGG3_CHUNKS = EDGES_PER_TILE_DEG // CHUNK
AGG3_TAIL = EDGES_PER_TILE_DEG - AGG3_CHUNKS * CHUNK


@functools.partial(
    pl.kernel,
    out_type=jax.ShapeDtypeStruct((NC, NP, HD3), jnp.float32),
    mesh=_mesh(),
    scratch_types=[
        pltpu.VMEM((3, 2, CHUNK), jnp.int32),
        pltpu.VMEM((3, CHUNK), jnp.int32),
        pltpu.VMEM((3, CHUNK), jnp.int32),
        pltpu.VMEM((3, CHUNK, HD3), jnp.float32),
        pltpu.VMEM((AGG3_TAIL,), jnp.int32),
        pltpu.VMEM((AGG3_TAIL,), jnp.int32),
        pltpu.VMEM((AGG3_TAIL, HD3), jnp.float32),
        pltpu.VMEM_SHARED((NP, HD3), jnp.float32),
        pltpu.SemaphoreType.DMA,
        pltpu.SemaphoreType.DMA,
        pltpu.SemaphoreType.DMA,
        pltpu.SemaphoreType.DMA,
        pltpu.SemaphoreType.DMA,
        pltpu.SemaphoreType.DMA,
        pltpu.SemaphoreType.DMA,
        pltpu.SemaphoreType.DMA,
        pltpu.SemaphoreType.DMA,
    ],
    compiler_params=pltpu.CompilerParams(use_tc_tiling_on_sc=False),
)
def _sc_agg64(hg_hbm, hh_hbm, ei_hbm, out_hbm,
              eiv, idxv, dstv, rowsv, srcv_t, dstv_t, rowsv_t,
              acc, se0, se1, se2, sg0, sg1, sg2, ss0, ss1, ss2):
    c = lax.axis_index("c")
    s = lax.axis_index("s")
    r0 = s * ROWS_PER_TILE
    sems_e = (se0, se1, se2)
    sems_g = (sg0, sg1, sg2)
    sems_s = (ss0, ss1, ss2)

    ebase = c * (E // NC) + s * EDGES_PER_TILE_DEG

    def fetch_ei(k, e):
        b = pl.multiple_of(ebase + k * CHUNK, 8)
        pltpu.async_copy(ei_hbm.at[:, pl.ds(b, CHUNK)], eiv.at[e], sems_e[e])

    def wait_scatter(g):
        pltpu.make_async_copy(rowsv.at[g], acc.at[dstv.at[g]],
                              sems_s[g]).wait()

    def start_gather(e, prior_scatter=None):
        pltpu.make_async_copy(ei_hbm.at[:, pl.ds(0, CHUNK)],
                              eiv.at[e], sems_e[e]).wait()
        if prior_scatter is not None:
            @pl.when(prior_scatter)
            def _():
                wait_scatter(e)
        for i in range(CHUNK // 16):
            sl = pl.ds(i * 16, 16)
            idxv[e, sl] = eiv[e, 0, sl]
            dstv[e, sl] = eiv[e, 1, sl]
        pltpu.async_copy(hg_hbm.at[idxv.at[e]], rowsv.at[e], sems_g[e])

    def scatter(g):
        pltpu.make_async_copy(hg_hbm.at[idxv.at[g]], rowsv.at[g],
                              sems_g[g]).wait()
        pltpu.async_copy(rowsv.at[g], acc.at[dstv.at[g]], sems_s[g], add=True)

    fetch_ei(0, 0)
    fetch_ei(1, 1)
    start_gather(0)

    pltpu.sync_copy(hh_hbm.at[pl.ds(r0, ROWS_PER_TILE)],
                    acc.at[pl.ds(r0, ROWS_PER_TILE)])
    plsc.subcore_barrier()

    @pl.loop(0, AGG3_CHUNKS, step=3)
    def _(j):
        for b in range(3):
            k = j + b

            @pl.when(k + 2 < AGG3_CHUNKS)
            def _():
                fetch_ei(k + 2, (b + 2) % 3)

            @pl.when(k + 1 < AGG3_CHUNKS)
            def _():
                start_gather((b + 1) % 3, prior_scatter=(k + 1 >= 3))

            scatter(b % 3)

    wait_scatter(0)
    wait_scatter(1)
    wait_scatter(2)

    bt = pl.multiple_of(ebase + AGG3_CHUNKS * CHUNK, 8)
    pltpu.sync_copy(ei_hbm.at[0, pl.ds(bt, AGG3_TAIL)], srcv_t)
    pltpu.sync_copy(ei_hbm.at[1, pl.ds(bt, AGG3_TAIL)], dstv_t)
    pltpu.async_copy(hg_hbm.at[srcv_t], rowsv_t, se0).wait()
    pltpu.sync_copy(rowsv_t, acc.at[dstv_t], add=True)

    plsc.subcore_barrier()
    pltpu.sync_copy(acc.at[pl.ds(r0, ROWS_PER_TILE)],
                    out_hbm.at[c, pl.ds(r0, ROWS_PER_TILE)])


def _split_store(out_ref, h):
    h = h.astype(out_ref.dtype)
    hw = h.shape[1] // 2
    out_ref[0:N, :] = h[:, 0:hw]
    out_ref[NP:NP + N, :] = h[:, hw:]
    pad = jnp.zeros((NP - N, hw), out_ref.dtype)
    out_ref[N:NP, :] = pad
    out_ref[NP + N:2 * NP, :] = pad


def _dinv(degp_ref):
    deg = degp_ref[0, 0:N, 0:1] + degp_ref[1, 0:N, 0:1] + 1.0
    return lax.rsqrt(deg)


def _tc1_body(degp_ref, x_ref, w_ref, out_ref):
    dinv = _dinv(degp_ref)
    h = jnp.dot(x_ref[...], w_ref[...], preferred_element_type=jnp.float32)
    h = h * dinv
    _split_store(out_ref, h)


def _tc_mid_body(degp_ref, agg_ref, b_ref, g_ref, be_ref, w_ref, out_ref):
    dinv = _dinv(degp_ref)
    z = jnp.concatenate([agg_ref[0:N, :], agg_ref[NP:NP + N, :]], axis=1)
    z = z.astype(jnp.float32) * dinv + b_ref[...]
    z = jnp.maximum(z, 0.0)
    mean = jnp.mean(z, axis=0, keepdims=True)
    var = jnp.mean((z - mean) * (z - mean), axis=0, keepdims=True)
    z = g_ref[...] * (z - mean) * lax.rsqrt(var + 1e-5) + be_ref[...]
    h = jnp.dot(z, w_ref[...], preferred_element_type=jnp.float32)
    h = h * dinv
    _split_store(out_ref, h)


def _tc3_body(degp_ref, agg_ref, b_ref, g_ref, be_ref, w_ref,
              outg_ref, outh_ref):
    dinv = _dinv(degp_ref)
    z = jnp.concatenate([agg_ref[0:N, :], agg_ref[NP:NP + N, :]], axis=1)
    z = z.astype(jnp.float32) * dinv + b_ref[...]
    z = jnp.maximum(z, 0.0)
    mean = jnp.mean(z, axis=0, keepdims=True)
    var = jnp.mean((z - mean) * (z - mean), axis=0, keepdims=True)
    z = g_ref[...] * (z - mean) * lax.rsqrt(var + 1e-5) + be_ref[...]
    h = jnp.dot(z, w_ref[...], preferred_element_type=jnp.float32)
    h = h * dinv
    pad = jnp.zeros((NP - N, HD3), jnp.float32)
    outg_ref[0:N, :] = h
    outg_ref[N:NP, :] = pad
    outh_ref[0:N, :] = 0.5 * h
    outh_ref[N:NP, :] = pad


def _tc_final_body(degp_ref, agg_ref, b_ref, out_ref):
    dinv = _dinv(degp_ref)
    z = agg_ref[0, 0:N, :] + agg_ref[1, 0:N, :]
    z = z * dinv + b_ref[...]
    m = jnp.max(z, axis=1, keepdims=True)
    zm = z - m
    lse = jnp.log(jnp.sum(jnp.exp(zm), axis=1, keepdims=True))
    out_ref[...] = zm - lse


def _tc_call(body, out_shape, *args, dtype=jnp.float32):
    return pl.pallas_call(
        body, out_shape=jax.ShapeDtypeStruct(out_shape, dtype))(*args)


def kernel(x, edge_index, W1, b1, W2, b2, W3, b3, gamma1, beta1, gamma2, beta2):
    src = edge_index[0]
    dst = edge_index[1]

    ones16 = jnp.ones((CHUNK, 16), jnp.float32)
    zeros16 = jnp.zeros((ROWS_PER_TILE, 16), jnp.float32)
    degp = _sc_degree(dst, ones16, zeros16)
    h1 = _tc_call(_tc1_body, (2 * NP, 128), degp, x, W1)
    a1 = _sc_agg128(h1, edge_index)
    h2 = _tc_call(_tc_mid_body, (2 * NP, 128), degp, a1,
                  b1.reshape(1, -1), gamma1.reshape(1, -1),
                  beta1.reshape(1, -1), W2)
    a2 = _sc_agg128(h2, edge_index)
    h3g, h3h = pl.pallas_call(
        _tc3_body,
        out_shape=[jax.ShapeDtypeStruct((NP, HD3), jnp.float32),
                   jax.ShapeDtypeStruct((NP, HD3), jnp.float32)],
    )(degp, a2, b2.reshape(1, -1), gamma2.reshape(1, -1),
      beta2.reshape(1, -1), W3)
    a3 = _sc_agg64(h3g, h3h, edge_index)
    out = _tc_call(_tc_final_body, (N, 64), degp, a3, b3.reshape(1, -1))
    return out
